# baseline (speedup 1.0000x reference)
import functools

import jax
import jax.numpy as jnp
from jax import lax
from jax.experimental import pallas as pl
from jax.experimental.pallas import tpu as pltpu

E = 8
K = 2
H = 1024
I = 2816
T = 2048
B = 128
NB = (T * K + E * (B - 1) + B - 1) // B
P = NB * B


def _routing_metadata(router_logits):
    topk_vals, topk_idx = lax.top_k(router_logits, K)
    topk_w = jax.nn.softmax(topk_vals, axis=-1)
    tok = jnp.arange(T, dtype=jnp.int32)
    mask = jnp.zeros((T, E), jnp.int32).at[tok[:, None], topk_idx].add(1)
    counts = mask.sum(axis=0)
    padded = ((counts + B - 1) // B) * B
    ends = jnp.cumsum(padded)
    starts = ends - padded
    pos = jnp.cumsum(mask, axis=0) - mask
    pos_k = jnp.take_along_axis(pos, topk_idx, axis=1)
    dest = starts[topk_idx] + pos_k
    flat_dest = dest.reshape(-1)
    gather_tok = jnp.zeros((P,), jnp.int32).at[flat_dest].set(
        jnp.broadcast_to(tok[:, None], (T, K)).reshape(-1))
    w_sorted = jnp.zeros((P,), jnp.float32).at[flat_dest].set(
        topk_w.reshape(-1))
    block_starts = jnp.arange(NB, dtype=jnp.int32) * B
    block_expert = jnp.minimum(
        jnp.sum(block_starts[:, None] >= ends[None, :], axis=1), E - 1
    ).astype(jnp.int32)
    block_valid = (block_starts < ends[-1]).astype(jnp.int32)
    return dest, gather_tok, w_sorted, block_expert, block_valid


def _ffn_body(be_ref, bv_ref, x_ref, w_ref, wg_ref, wu_ref, wd_ref, out_ref):
    b = pl.program_id(0)

    @pl.when(bv_ref[b] == 1)
    def _():
        x = x_ref[...].astype(jnp.bfloat16)
        g = lax.dot_general(x, wg_ref[0], (((1,), (1,)), ((), ())),
                            preferred_element_type=jnp.float32)
        u = lax.dot_general(x, wu_ref[0], (((1,), (1,)), ((), ())),
                            preferred_element_type=jnp.float32)
        h = (g * jax.nn.sigmoid(g) * u).astype(jnp.bfloat16)
        y = lax.dot_general(h, wd_ref[0], (((1,), (1,)), ((), ())),
                            preferred_element_type=jnp.float32)
        out_ref[...] = y * w_ref[...]


def _expert_ffn(x_sorted, w_sorted, block_expert, block_valid,
                wg, wu, wd):
    grid_spec = pltpu.PrefetchScalarGridSpec(
        num_scalar_prefetch=2,
        grid=(NB,),
        in_specs=[
            pl.BlockSpec((B, H), lambda b, be, bv: (b, 0)),
            pl.BlockSpec((B, 1), lambda b, be, bv: (b, 0)),
            pl.BlockSpec((1, I, H), lambda b, be, bv: (be[b], 0, 0)),
            pl.BlockSpec((1, I, H), lambda b, be, bv: (be[b], 0, 0)),
            pl.BlockSpec((1, H, I), lambda b, be, bv: (be[b], 0, 0)),
        ],
        out_specs=pl.BlockSpec((B, H), lambda b, be, bv: (b, 0)),
    )
    return pl.pallas_call(
        _ffn_body,
        grid_spec=grid_spec,
        out_shape=jax.ShapeDtypeStruct((P, H), jnp.float32),
        compiler_params=pltpu.CompilerParams(
            dimension_semantics=("arbitrary",)),
    )(block_expert, block_valid, x_sorted, w_sorted[:, None], wg, wu, wd)


def kernel(hidden_states, router_logits, W_gate, W_up, W_down):
    dest, gather_tok, w_sorted, block_expert, block_valid = (
        _routing_metadata(router_logits))
    x_sorted = jnp.take(hidden_states, gather_tok, axis=0)
    ys = _expert_ffn(x_sorted, w_sorted, block_expert, block_valid,
                     W_gate.astype(jnp.bfloat16),
                     W_up.astype(jnp.bfloat16),
                     W_down.astype(jnp.bfloat16))
    y = jnp.take(ys, dest[:, 0], axis=0) + jnp.take(ys, dest[:, 1], axis=0)
    return y
